# Initial kernel scaffold; baseline (speedup 1.0000x reference)
import jax
import jax.numpy as jnp
from jax.experimental import pallas as pl


def kernel(z, edge_index):
    raise NotImplementedError("write your pallas kernel here")


# speedup vs baseline: 1.3369x; 1.3369x over previous
import functools

import jax
import jax.numpy as jnp
from jax import lax
from jax.experimental import pallas as pl
from jax.experimental.pallas import tpu as pltpu
from jax.experimental.pallas import tpu_sc as plsc

_D = 128
_E = 320000
_NC = 2
_NS = 16
_NW = _NC * _NS
_EW = _E // _NW
_C = 80
_NCH = _EW // _C
_G = _C // 16
_NBUF = 4


def _ipd_body(z_hbm, row_hbm, col_hbm, out_hbm,
              idxr_v, idxc_v, rr_v, rc_v, out_v, gsems):
    cid = lax.axis_index("c")
    sid = lax.axis_index("s")
    wid = sid * _NC + cid
    wbase = wid * _EW

    iota = lax.iota(jnp.int32, 16)

    pltpu.sync_copy(row_hbm.at[pl.ds(wbase, _EW)], idxr_v)
    pltpu.sync_copy(col_hbm.at[pl.ds(wbase, _EW)], idxc_v)

    def start(i, b):
        idxr = idxr_v.at[pl.ds(i * _C, _C)]
        idxc = idxc_v.at[pl.ds(i * _C, _C)]
        pltpu.async_copy(z_hbm.at[idxr], rr_v.at[b], gsems.at[b])
        pltpu.async_copy(z_hbm.at[idxc], rc_v.at[b], gsems.at[b])

    def finish(i, b):
        idxr = idxr_v.at[pl.ds(i * _C, _C)]
        idxc = idxc_v.at[pl.ds(i * _C, _C)]
        pltpu.make_async_copy(z_hbm.at[idxr], rr_v.at[b], gsems.at[b]).wait()
        pltpu.make_async_copy(z_hbm.at[idxc], rc_v.at[b], gsems.at[b]).wait()
        for g in range(_G):
            e16 = g * 16 + iota

            def dbody(d, acc):
                dsp = jnp.full((16,), d, jnp.int32)
                a = plsc.load_gather(rr_v.at[b], [e16, dsp])
                bb = plsc.load_gather(rc_v.at[b], [e16, dsp])
                return acc + a * bb

            acc = lax.fori_loop(0, _D, dbody, jnp.zeros((16,), jnp.float32),
                                unroll=8)
            out_v[pl.ds(i * _C + g * 16, 16)] = 1.0 / (1.0 + jnp.exp(-acc))

    for b in range(_NBUF - 1):
        start(b, b)

    def outer(j, carry):
        i0 = j * _NBUF
        for b in range(_NBUF):
            i = i0 + b
            nxt = i + _NBUF - 1

            @pl.when(nxt < _NCH)
            def _():
                start(nxt, (b + _NBUF - 1) % _NBUF)

            @pl.when(i < _NCH)
            def _():
                finish(i, b)
        return carry

    lax.fori_loop(0, (_NCH + _NBUF - 1) // _NBUF, outer, 0)

    pltpu.sync_copy(out_v, out_hbm.at[pl.ds(wbase, _EW)])


@jax.jit
def kernel(z, edge_index):
    ei = edge_index.astype(jnp.int32)
    row = ei[0]
    col = ei[1]
    mesh = plsc.VectorSubcoreMesh(
        core_axis_name="c", subcore_axis_name="s",
        num_cores=_NC, num_subcores=_NS)
    f = pl.kernel(
        _ipd_body,
        out_type=jax.ShapeDtypeStruct((_E,), jnp.float32),
        mesh=mesh,
        scratch_types=[
            pltpu.VMEM((_EW,), jnp.int32),
            pltpu.VMEM((_EW,), jnp.int32),
            pltpu.VMEM((_NBUF, _C, _D), jnp.float32),
            pltpu.VMEM((_NBUF, _C, _D), jnp.float32),
            pltpu.VMEM((_EW,), jnp.float32),
            pltpu.SemaphoreType.DMA((_NBUF,)),
        ],
        compiler_params=pltpu.CompilerParams(needs_layout_passes=False),
    )
    return f(z, row, col)
